# baseline (speedup 1.0000x reference)
import functools

import jax
import jax.numpy as jnp
from jax import lax
from jax.experimental import pallas as pl
from jax.experimental.pallas import tpu as pltpu
from jax.experimental.pallas import tpu_sc as plsc

N_NODES = 10000
N_EDGES = 320000
HID = 128
HP = 16
NC = 2
NS = 16
NW = NC * NS
EPW = N_EDGES // NW
BLK = 40
NBLK = EPW // BLK
PIPE = 5
NBODY = NBLK // PIPE
PIPE_C = 2
NBODY_C = NBLK // PIPE_C
NP = 10240
ROWS_PT = NP // NS

LAYER_CFG = [(16, 8, True), (16, 8, True), (128, 1, False)]


def _node_proj_body(x_ref, w_ref, b_ref, o_ref):
  o_ref[...] = (
      jnp.dot(x_ref[...], w_ref[...], preferred_element_type=jnp.float32)
      + b_ref[...]
  )


def _node_proj(x, w, b):
  return pl.pallas_call(
      _node_proj_body,
      out_shape=jax.ShapeDtypeStruct((N_NODES, HID), jnp.float32),
  )(x, w, b[None, :])


_EB = 8000


def _edge_logits_body(ea_ref, p_ref, c_ref, o0, o1, o2, m0, m1, m2):
  z = (
      jnp.dot(ea_ref[...], p_ref[...], preferred_element_type=jnp.float32)
      + c_ref[...]
  )
  i = pl.program_id(0)
  for k, (o_ref, m_ref) in enumerate(((o0, m0), (o1, m1), (o2, m2))):
    a = z[:, 16 * k:16 * (k + 1)]
    o_ref[...] = a
    bm = jnp.max(a, axis=0, keepdims=True)

    @pl.when(i == 0)
    def _():
      m_ref[...] = bm

    @pl.when(i != 0)
    def _():
      m_ref[...] = jnp.maximum(m_ref[...], bm)


def _edge_logits(edge_attr, p_all, c_all):
  n_steps = N_EDGES // _EB
  ae_shape = jax.ShapeDtypeStruct((N_EDGES, HP), jnp.float32)
  mx_shape = jax.ShapeDtypeStruct((1, HP), jnp.float32)
  return pl.pallas_call(
      _edge_logits_body,
      grid=(n_steps,),
      in_specs=[
          pl.BlockSpec((_EB, 16), lambda i: (i, 0)),
          pl.BlockSpec((16, 3 * HP), lambda i: (0, 0)),
          pl.BlockSpec((1, 3 * HP), lambda i: (0, 0)),
      ],
      out_specs=[
          pl.BlockSpec((_EB, HP), lambda i: (i, 0)),
          pl.BlockSpec((_EB, HP), lambda i: (i, 0)),
          pl.BlockSpec((_EB, HP), lambda i: (i, 0)),
          pl.BlockSpec((1, HP), lambda i: (0, 0)),
          pl.BlockSpec((1, HP), lambda i: (0, 0)),
          pl.BlockSpec((1, HP), lambda i: (0, 0)),
      ],
      out_shape=[ae_shape, ae_shape, ae_shape, mx_shape, mx_shape, mx_shape],
  )(edge_attr, p_all, c_all[None, :])


def _layer_dense_body(h_ref, w_ref, ss_ref, sd_ref,
                      hw_ref, as_ref, ad_ref, ms_ref, md_ref):
  hw = jnp.dot(h_ref[...], w_ref[...], preferred_element_type=jnp.float32)
  hw_ref[...] = hw
  a_s = jnp.dot(hw, ss_ref[...], preferred_element_type=jnp.float32)
  a_d = jnp.dot(hw, sd_ref[...], preferred_element_type=jnp.float32)
  as_ref[...] = a_s
  ad_ref[...] = a_d
  ms_ref[...] = jnp.max(a_s, axis=0, keepdims=True)
  md_ref[...] = jnp.max(a_d, axis=0, keepdims=True)


def _layer_dense(h, w, s_src, s_dst):
  return pl.pallas_call(
      _layer_dense_body,
      out_shape=[
          jax.ShapeDtypeStruct((N_NODES, HID), jnp.float32),
          jax.ShapeDtypeStruct((N_NODES, HP), jnp.float32),
          jax.ShapeDtypeStruct((N_NODES, HP), jnp.float32),
          jax.ShapeDtypeStruct((1, HP), jnp.float32),
          jax.ShapeDtypeStruct((1, HP), jnp.float32),
      ],
  )(h, w, s_src, s_dst)


def _epilogue_body(agg_ref, b_ref, g_ref, be_ref, res_ref, o_ref, *, relu):
  hn = agg_ref[0, :N_NODES] + agg_ref[1, :N_NODES] + b_ref[...]
  mu = jnp.mean(hn, axis=0, keepdims=True)
  var = jnp.mean((hn - mu) * (hn - mu), axis=0, keepdims=True)
  hn = (hn - mu) * lax.rsqrt(var + 1e-5) * g_ref[...] + be_ref[...]
  if relu:
    hn = jnp.maximum(hn, 0.0)
  o_ref[...] = hn + res_ref[...]


def _epilogue(agg_parts, bias, gamma, beta, residual, relu):
  return pl.pallas_call(
      functools.partial(_epilogue_body, relu=relu),
      out_shape=jax.ShapeDtypeStruct((N_NODES, HID), jnp.float32),
  )(agg_parts, bias[None, :], gamma[None, :], beta[None, :], residual)


def _epi_dense_body(agg_ref, b_ref, g_ref, be_ref, res_ref,
                    w_ref, ss_ref, sd_ref,
                    h_ref, hw_ref, as_ref, ad_ref, ms_ref, md_ref):
  hn = agg_ref[0, :N_NODES] + agg_ref[1, :N_NODES] + b_ref[...]
  mu = jnp.mean(hn, axis=0, keepdims=True)
  var = jnp.mean((hn - mu) * (hn - mu), axis=0, keepdims=True)
  hn = (hn - mu) * lax.rsqrt(var + 1e-5) * g_ref[...] + be_ref[...]
  hn = jnp.maximum(hn, 0.0)
  h = hn + res_ref[...]
  h_ref[...] = h
  hw = jnp.dot(h, w_ref[...], preferred_element_type=jnp.float32)
  hw_ref[...] = hw
  a_s = jnp.dot(hw, ss_ref[...], preferred_element_type=jnp.float32)
  a_d = jnp.dot(hw, sd_ref[...], preferred_element_type=jnp.float32)
  as_ref[...] = a_s
  ad_ref[...] = a_d
  ms_ref[...] = jnp.max(a_s, axis=0, keepdims=True)
  md_ref[...] = jnp.max(a_d, axis=0, keepdims=True)


def _epi_dense(agg_parts, bias, gamma, beta, residual, w, s_src, s_dst):
  return pl.pallas_call(
      _epi_dense_body,
      out_shape=[
          jax.ShapeDtypeStruct((N_NODES, HID), jnp.float32),
          jax.ShapeDtypeStruct((N_NODES, HID), jnp.float32),
          jax.ShapeDtypeStruct((N_NODES, HP), jnp.float32),
          jax.ShapeDtypeStruct((N_NODES, HP), jnp.float32),
          jax.ShapeDtypeStruct((1, HP), jnp.float32),
          jax.ShapeDtypeStruct((1, HP), jnp.float32),
      ],
  )(agg_parts, bias[None, :], gamma[None, :], beta[None, :], residual,
    w, s_src, s_dst)


def _proj_dense_body(x_ref, wp_ref, bp_ref, w_ref, ss_ref, sd_ref,
                     h_ref, hw_ref, as_ref, ad_ref, ms_ref, md_ref):
  h = (jnp.dot(x_ref[...], wp_ref[...], preferred_element_type=jnp.float32)
       + bp_ref[...])
  h_ref[...] = h
  hw = jnp.dot(h, w_ref[...], preferred_element_type=jnp.float32)
  hw_ref[...] = hw
  a_s = jnp.dot(hw, ss_ref[...], preferred_element_type=jnp.float32)
  a_d = jnp.dot(hw, sd_ref[...], preferred_element_type=jnp.float32)
  as_ref[...] = a_s
  ad_ref[...] = a_d
  ms_ref[...] = jnp.max(a_s, axis=0, keepdims=True)
  md_ref[...] = jnp.max(a_d, axis=0, keepdims=True)


def _proj_dense(x, wp, bp, w, s_src, s_dst):
  return pl.pallas_call(
      _proj_dense_body,
      out_shape=[
          jax.ShapeDtypeStruct((N_NODES, HID), jnp.float32),
          jax.ShapeDtypeStruct((N_NODES, HID), jnp.float32),
          jax.ShapeDtypeStruct((N_NODES, HP), jnp.float32),
          jax.ShapeDtypeStruct((N_NODES, HP), jnp.float32),
          jax.ShapeDtypeStruct((1, HP), jnp.float32),
          jax.ShapeDtypeStruct((1, HP), jnp.float32),
      ],
  )(x, wp, bp[None, :], w, s_src, s_dst)


_MESH = plsc.VectorSubcoreMesh(core_axis_name="c", subcore_axis_name="s")


@functools.partial(
    pl.kernel,
    mesh=_MESH,
    out_type=[
        jax.ShapeDtypeStruct((N_EDGES, HP), jnp.float32),
        jax.ShapeDtypeStruct((NC, NP, HP), jnp.float32),
    ],
    compiler_params=pltpu.CompilerParams(use_tc_tiling_on_sc=False),
    scratch_types=(
        [pltpu.VMEM((NBLK, BLK), jnp.int32) for _ in range(2)]
        + [pltpu.VMEM((BLK, HP), jnp.float32) for _ in range(4 * PIPE)]
        + [pltpu.VMEM((16,), jnp.float32),
           pltpu.VMEM_SHARED((NP, HP), jnp.float32)]
        + [pltpu.SemaphoreType.DMA for _ in range(5 * PIPE)]
    ),
)
def _sc_pass_a(src_h, dst_h, asrc_h, adst_h, ae_h, mv_h, z16_h,
               ex_h, den_h, *refs):
  srcall, dstall = refs[0], refs[1]
  asr = refs[2:2 + PIPE]
  adr = refs[2 + PIPE:2 + 2 * PIPE]
  aer = refs[2 + 2 * PIPE:2 + 3 * PIPE]
  exr = refs[2 + 3 * PIPE:2 + 4 * PIPE]
  mv = refs[2 + 4 * PIPE]
  den_sp = refs[3 + 4 * PIPE]
  sems = refs[4 + 4 * PIPE:]
  gsem = [sems[5 * j:5 * j + 3] for j in range(PIPE)]
  osem = [sems[5 * j + 3:5 * j + 5] for j in range(PIPE)]

  cid = lax.axis_index("c")
  sid = lax.axis_index("s")
  wid = cid * NS + sid
  pltpu.sync_copy(z16_h.at[pl.ds(sid * ROWS_PT, ROWS_PT)],
                  den_sp.at[pl.ds(sid * ROWS_PT, ROWS_PT)])
  pltpu.sync_copy(mv_h, mv)
  pltpu.sync_copy(src_h.at[wid], srcall)
  pltpu.sync_copy(dst_h.at[wid], dstall)
  plsc.subcore_barrier()
  mrow = mv[...]
  base0 = wid * EPW

  @pl.loop(0, NBODY)
  def _body(k):
    blk0 = k * PIPE
    gds = []
    for j in range(PIPE):
      blk = blk0 + j
      gds.append((
          pltpu.async_copy(asrc_h.at[srcall.at[blk]], asr[j], gsem[j][0]),
          pltpu.async_copy(adst_h.at[dstall.at[blk]], adr[j], gsem[j][1]),
          pltpu.async_copy(ae_h.at[pl.ds(base0 + blk * BLK, BLK)], aer[j],
                           gsem[j][2]),
      ))
    ods = []
    for j in range(PIPE):
      blk = blk0 + j
      for g in gds[j]:
        g.wait()

      @pl.loop(0, BLK)
      def _edge(e):
        a = asr[j][e, :] + adr[j][e, :] + aer[j][e, :]
        a = jnp.where(a < 0.0, a * 0.2, a)
        exr[j][e, :] = jnp.exp(a - mrow)

      ods.append(pltpu.async_copy(
          exr[j], ex_h.at[pl.ds(base0 + blk * BLK, BLK)], osem[j][0]))
      ods.append(pltpu.async_copy(
          exr[j], den_sp.at[dstall.at[blk]], osem[j][1], add=True))
    for o in ods:
      o.wait()

  plsc.subcore_barrier()
  pltpu.sync_copy(den_sp.at[pl.ds(sid * ROWS_PT, ROWS_PT)],
                  den_h.at[cid].at[pl.ds(sid * ROWS_PT, ROWS_PT)])


def _make_sc_pass_c(heads):
  @functools.partial(
      pl.kernel,
      mesh=_MESH,
      out_type=jax.ShapeDtypeStruct((NC, NP, HID), jnp.float32),
      compiler_params=pltpu.CompilerParams(use_tc_tiling_on_sc=False),
      scratch_types=(
          [pltpu.VMEM((NBLK, BLK), jnp.int32) for _ in range(2)]
          + [pltpu.VMEM((BLK, HP), jnp.float32) for _ in range(3 * PIPE_C)]
          + [pltpu.VMEM((BLK, HID), jnp.float32) for _ in range(PIPE_C)]
          + [pltpu.VMEM_SHARED((NP, HID), jnp.float32)]
          + [pltpu.SemaphoreType.DMA for _ in range(5 * PIPE_C)]
      ),
  )
  def _sc_pass_c(src_h, dst_h, ex_h, den_h, hw_h, z128_h,
                 agg_h, *refs):
    srcall, dstall = refs[0], refs[1]
    exr = refs[2:2 + PIPE_C]
    dv0 = refs[2 + PIPE_C:2 + 2 * PIPE_C]
    dv1 = refs[2 + 2 * PIPE_C:2 + 3 * PIPE_C]
    rows = refs[2 + 3 * PIPE_C:2 + 4 * PIPE_C]
    agg_sp = refs[2 + 4 * PIPE_C]
    sems = refs[3 + 4 * PIPE_C:]
    gsem = [sems[5 * j:5 * j + 4] for j in range(PIPE_C)]
    osem = [sems[5 * j + 4] for j in range(PIPE_C)]

    cid = lax.axis_index("c")
    sid = lax.axis_index("s")
    wid = cid * NS + sid
    pltpu.sync_copy(z128_h.at[pl.ds(sid * ROWS_PT, ROWS_PT)],
                    agg_sp.at[pl.ds(sid * ROWS_PT, ROWS_PT)])
    pltpu.sync_copy(src_h.at[wid], srcall)
    pltpu.sync_copy(dst_h.at[wid], dstall)
    plsc.subcore_barrier()
    base0 = wid * EPW

    @pl.loop(0, NBODY_C)
    def _body(k):
      blk0 = k * PIPE_C
      gds = []
      for j in range(PIPE_C):
        blk = blk0 + j
        gds.append((
            pltpu.async_copy(ex_h.at[pl.ds(base0 + blk * BLK, BLK)], exr[j],
                             gsem[j][0]),
            pltpu.async_copy(den_h.at[0].at[dstall.at[blk]], dv0[j],
                             gsem[j][1]),
            pltpu.async_copy(den_h.at[1].at[dstall.at[blk]], dv1[j],
                             gsem[j][2]),
            pltpu.async_copy(hw_h.at[srcall.at[blk]], rows[j], gsem[j][3]),
        ))
      ods = []
      for j in range(PIPE_C):
        blk = blk0 + j
        for g in gds[j]:
          g.wait()

        @pl.loop(0, BLK)
        def _edge(e):
          crow = exr[j][e, :] / (dv0[j][e, :] + dv1[j][e, :] + 1e-16)
          if heads > 1:
            for h in range(heads):
              sc = jnp.take(crow, jnp.full((16,), h, jnp.int32))
              rows[j][e, pl.ds(h * 16, 16)] = rows[j][e, pl.ds(h * 16, 16)] * sc
          else:
            sc = jnp.take(crow, jnp.full((16,), 0, jnp.int32))
            for q in range(HID // 16):
              rows[j][e, pl.ds(q * 16, 16)] = rows[j][e, pl.ds(q * 16, 16)] * sc

        ods.append(pltpu.async_copy(
            rows[j], agg_sp.at[dstall.at[blk]], osem[j], add=True))
      for o in ods:
        o.wait()

    plsc.subcore_barrier()
    pltpu.sync_copy(agg_sp.at[pl.ds(sid * ROWS_PT, ROWS_PT)],
                    agg_h.at[cid].at[pl.ds(sid * ROWS_PT, ROWS_PT)])

  return _sc_pass_c


_SC_PASS_C = {h: _make_sc_pass_c(h) for h in (8, 1)}


def _fold_params(params):
  folded = {}
  p_cols = []
  c_cols = []
  for i, (out, heads, _) in enumerate(LAYER_CFG):
    we = params[f'l{i}_We'].reshape(HID, heads, out)
    m = jnp.einsum('khc,hc->kh', we, params[f'l{i}_att_edge'])
    p_pad = jnp.zeros((16, HP), jnp.float32)
    p_pad = p_pad.at[:, :heads].set(params['edge_proj_W'] @ m)
    c_pad = jnp.zeros((HP,), jnp.float32)
    c_pad = c_pad.at[:heads].set(params['edge_proj_b'] @ m)
    p_cols.append(p_pad)
    c_cols.append(c_pad)

    eye = jnp.eye(heads, dtype=jnp.float32)
    for name in ('src', 'dst'):
      att = params[f'l{i}_att_{name}']
      s = (att[:, :, None] * eye[:, None, :]).reshape(heads * out, heads)
      s_pad = jnp.zeros((HID, HP), jnp.float32)
      s_pad = s_pad.at[:, :heads].set(s)
      folded[f'l{i}_S{name}'] = s_pad
  folded['P_all'] = jnp.concatenate(p_cols, axis=1)
  folded['c_all'] = jnp.concatenate(c_cols, axis=0)
  return folded


def kernel(x, edge_index, edge_attr, params):
  src = edge_index[0].astype(jnp.int32).reshape(NW, NBLK, BLK)
  dst = edge_index[1].astype(jnp.int32).reshape(NW, NBLK, BLK)
  folded = _fold_params(params)

  ae0, ae1, ae2, am0, am1, am2 = _edge_logits(
      edge_attr, folded['P_all'], folded['c_all'])
  ae_list = (ae0, ae1, ae2)
  ae_max = (am0, am1, am2)

  zeros16 = jnp.zeros((NP, HP), jnp.float32)
  zeros128 = jnp.zeros((NP, HID), jnp.float32)

  h, hw, asrc, adst, ms, md = _proj_dense(
      x, params['node_proj_W'], params['node_proj_b'],
      params['l0_W'], folded['l0_Ssrc'], folded['l0_Sdst'])

  for i, (out, heads, concat) in enumerate(LAYER_CFG):
    mvec = jnp.maximum(ms[0] + md[0] + ae_max[i][0], 0.0)
    ex, den_parts = _sc_pass_a(src, dst, asrc, adst, ae_list[i],
                               mvec, zeros16)
    agg_parts = _SC_PASS_C[heads](src, dst, ex, den_parts, hw, zeros128)
    if i < len(LAYER_CFG) - 1:
      h, hw, asrc, adst, ms, md = _epi_dense(
          agg_parts, params[f'l{i}_bias'], params[f'l{i}_bn_gamma'],
          params[f'l{i}_bn_beta'], h,
          params[f'l{i+1}_W'], folded[f'l{i+1}_Ssrc'],
          folded[f'l{i+1}_Sdst'])
    else:
      h = _epilogue(agg_parts, params[f'l{i}_bias'],
                    params[f'l{i}_bn_gamma'], params[f'l{i}_bn_beta'], h,
                    relu=False)
  return h


# speedup vs baseline: 1.0979x; 1.0979x over previous
import functools

import jax
import jax.numpy as jnp
from jax import lax
from jax.experimental import pallas as pl
from jax.experimental.pallas import tpu as pltpu
from jax.experimental.pallas import tpu_sc as plsc

N_NODES = 10000
N_EDGES = 320000
HID = 128
HP = 16
NC = 2
NS = 16
NW = NC * NS
EPW = N_EDGES // NW
BLK = 40
NBLK = EPW // BLK
PIPE = 5
NBODY = NBLK // PIPE
PIPE_C = 2
NBODY_C = NBLK // PIPE_C
NP = 10240
ROWS_PT = NP // NS

LAYER_CFG = [(16, 8, True), (16, 8, True), (128, 1, False)]


def _node_proj_body(x_ref, w_ref, b_ref, o_ref):
  o_ref[...] = (
      jnp.dot(x_ref[...], w_ref[...], preferred_element_type=jnp.float32)
      + b_ref[...]
  )


def _node_proj(x, w, b):
  return pl.pallas_call(
      _node_proj_body,
      out_shape=jax.ShapeDtypeStruct((N_NODES, HID), jnp.float32),
  )(x, w, b[None, :])


_EB = 8000


def _edge_logits_body(ea_ref, p_ref, c_ref, o0, o1, o2, m0, m1, m2):
  z = (
      jnp.dot(ea_ref[...], p_ref[...], preferred_element_type=jnp.float32)
      + c_ref[...]
  )
  i = pl.program_id(0)
  for k, (o_ref, m_ref) in enumerate(((o0, m0), (o1, m1), (o2, m2))):
    a = z[:, 16 * k:16 * (k + 1)]
    o_ref[...] = a
    bm = jnp.max(a, axis=0, keepdims=True)

    @pl.when(i == 0)
    def _():
      m_ref[...] = bm

    @pl.when(i != 0)
    def _():
      m_ref[...] = jnp.maximum(m_ref[...], bm)


def _edge_logits(edge_attr, p_all, c_all):
  n_steps = N_EDGES // _EB
  ae_shape = jax.ShapeDtypeStruct((N_EDGES, HP), jnp.float32)
  mx_shape = jax.ShapeDtypeStruct((1, HP), jnp.float32)
  return pl.pallas_call(
      _edge_logits_body,
      grid=(n_steps,),
      in_specs=[
          pl.BlockSpec((_EB, 16), lambda i: (i, 0)),
          pl.BlockSpec((16, 3 * HP), lambda i: (0, 0)),
          pl.BlockSpec((1, 3 * HP), lambda i: (0, 0)),
      ],
      out_specs=[
          pl.BlockSpec((_EB, HP), lambda i: (i, 0)),
          pl.BlockSpec((_EB, HP), lambda i: (i, 0)),
          pl.BlockSpec((_EB, HP), lambda i: (i, 0)),
          pl.BlockSpec((1, HP), lambda i: (0, 0)),
          pl.BlockSpec((1, HP), lambda i: (0, 0)),
          pl.BlockSpec((1, HP), lambda i: (0, 0)),
      ],
      out_shape=[ae_shape, ae_shape, ae_shape, mx_shape, mx_shape, mx_shape],
  )(edge_attr, p_all, c_all[None, :])


def _layer_dense_body(h_ref, w_ref, ss_ref, sd_ref,
                      hw_ref, as_ref, ad_ref, ms_ref, md_ref):
  hw = jnp.dot(h_ref[...], w_ref[...], preferred_element_type=jnp.float32)
  hw_ref[...] = hw
  a_s = jnp.dot(hw, ss_ref[...], preferred_element_type=jnp.float32)
  a_d = jnp.dot(hw, sd_ref[...], preferred_element_type=jnp.float32)
  as_ref[...] = a_s
  ad_ref[...] = a_d
  ms_ref[...] = jnp.max(a_s, axis=0, keepdims=True)
  md_ref[...] = jnp.max(a_d, axis=0, keepdims=True)


def _layer_dense(h, w, s_src, s_dst):
  return pl.pallas_call(
      _layer_dense_body,
      out_shape=[
          jax.ShapeDtypeStruct((N_NODES, HID), jnp.float32),
          jax.ShapeDtypeStruct((N_NODES, HP), jnp.float32),
          jax.ShapeDtypeStruct((N_NODES, HP), jnp.float32),
          jax.ShapeDtypeStruct((1, HP), jnp.float32),
          jax.ShapeDtypeStruct((1, HP), jnp.float32),
      ],
  )(h, w, s_src, s_dst)


def _inv_denom_body(d_ref, o_ref):
  o_ref[...] = 1.0 / (d_ref[0] + d_ref[1] + 1e-16)


def _inv_denom(den_parts):
  return pl.pallas_call(
      _inv_denom_body,
      out_shape=jax.ShapeDtypeStruct((NP, HP), jnp.float32),
  )(den_parts)


def _epilogue_body(agg_ref, b_ref, g_ref, be_ref, res_ref, o_ref, *, relu):
  hn = agg_ref[0, :N_NODES] + agg_ref[1, :N_NODES] + b_ref[...]
  mu = jnp.mean(hn, axis=0, keepdims=True)
  var = jnp.mean((hn - mu) * (hn - mu), axis=0, keepdims=True)
  hn = (hn - mu) * lax.rsqrt(var + 1e-5) * g_ref[...] + be_ref[...]
  if relu:
    hn = jnp.maximum(hn, 0.0)
  o_ref[...] = hn + res_ref[...]


def _epilogue(agg_parts, bias, gamma, beta, residual, relu):
  return pl.pallas_call(
      functools.partial(_epilogue_body, relu=relu),
      out_shape=jax.ShapeDtypeStruct((N_NODES, HID), jnp.float32),
  )(agg_parts, bias[None, :], gamma[None, :], beta[None, :], residual)


def _epi_dense_body(agg_ref, b_ref, g_ref, be_ref, res_ref,
                    w_ref, ss_ref, sd_ref,
                    h_ref, hw_ref, as_ref, ad_ref, ms_ref, md_ref):
  hn = agg_ref[0, :N_NODES] + agg_ref[1, :N_NODES] + b_ref[...]
  mu = jnp.mean(hn, axis=0, keepdims=True)
  var = jnp.mean((hn - mu) * (hn - mu), axis=0, keepdims=True)
  hn = (hn - mu) * lax.rsqrt(var + 1e-5) * g_ref[...] + be_ref[...]
  hn = jnp.maximum(hn, 0.0)
  h = hn + res_ref[...]
  h_ref[...] = h
  hw = jnp.dot(h, w_ref[...], preferred_element_type=jnp.float32)
  hw_ref[...] = hw
  a_s = jnp.dot(hw, ss_ref[...], preferred_element_type=jnp.float32)
  a_d = jnp.dot(hw, sd_ref[...], preferred_element_type=jnp.float32)
  as_ref[...] = a_s
  ad_ref[...] = a_d
  ms_ref[...] = jnp.max(a_s, axis=0, keepdims=True)
  md_ref[...] = jnp.max(a_d, axis=0, keepdims=True)


def _epi_dense(agg_parts, bias, gamma, beta, residual, w, s_src, s_dst):
  return pl.pallas_call(
      _epi_dense_body,
      out_shape=[
          jax.ShapeDtypeStruct((N_NODES, HID), jnp.float32),
          jax.ShapeDtypeStruct((N_NODES, HID), jnp.float32),
          jax.ShapeDtypeStruct((N_NODES, HP), jnp.float32),
          jax.ShapeDtypeStruct((N_NODES, HP), jnp.float32),
          jax.ShapeDtypeStruct((1, HP), jnp.float32),
          jax.ShapeDtypeStruct((1, HP), jnp.float32),
      ],
  )(agg_parts, bias[None, :], gamma[None, :], beta[None, :], residual,
    w, s_src, s_dst)


def _proj_dense_body(x_ref, wp_ref, bp_ref, w_ref, ss_ref, sd_ref,
                     h_ref, hw_ref, as_ref, ad_ref, ms_ref, md_ref):
  h = (jnp.dot(x_ref[...], wp_ref[...], preferred_element_type=jnp.float32)
       + bp_ref[...])
  h_ref[...] = h
  hw = jnp.dot(h, w_ref[...], preferred_element_type=jnp.float32)
  hw_ref[...] = hw
  a_s = jnp.dot(hw, ss_ref[...], preferred_element_type=jnp.float32)
  a_d = jnp.dot(hw, sd_ref[...], preferred_element_type=jnp.float32)
  as_ref[...] = a_s
  ad_ref[...] = a_d
  ms_ref[...] = jnp.max(a_s, axis=0, keepdims=True)
  md_ref[...] = jnp.max(a_d, axis=0, keepdims=True)


def _proj_dense(x, wp, bp, w, s_src, s_dst):
  return pl.pallas_call(
      _proj_dense_body,
      out_shape=[
          jax.ShapeDtypeStruct((N_NODES, HID), jnp.float32),
          jax.ShapeDtypeStruct((N_NODES, HID), jnp.float32),
          jax.ShapeDtypeStruct((N_NODES, HP), jnp.float32),
          jax.ShapeDtypeStruct((N_NODES, HP), jnp.float32),
          jax.ShapeDtypeStruct((1, HP), jnp.float32),
          jax.ShapeDtypeStruct((1, HP), jnp.float32),
      ],
  )(x, wp, bp[None, :], w, s_src, s_dst)


_MESH = plsc.VectorSubcoreMesh(core_axis_name="c", subcore_axis_name="s")


@functools.partial(
    pl.kernel,
    mesh=_MESH,
    out_type=[
        jax.ShapeDtypeStruct((N_EDGES, HP), jnp.float32),
        jax.ShapeDtypeStruct((NC, NP, HP), jnp.float32),
    ],
    compiler_params=pltpu.CompilerParams(use_tc_tiling_on_sc=False),
    scratch_types=(
        [pltpu.VMEM((NBLK, BLK), jnp.int32) for _ in range(2)]
        + [pltpu.VMEM((BLK, HP), jnp.float32) for _ in range(4 * PIPE)]
        + [pltpu.VMEM((16,), jnp.float32),
           pltpu.VMEM_SHARED((NP, HP), jnp.float32)]
        + [pltpu.SemaphoreType.DMA for _ in range(5 * PIPE)]
    ),
)
def _sc_pass_a(src_h, dst_h, asrc_h, adst_h, ae_h, mv_h, z16_h,
               ex_h, den_h, *refs):
  srcall, dstall = refs[0], refs[1]
  asr = refs[2:2 + PIPE]
  adr = refs[2 + PIPE:2 + 2 * PIPE]
  aer = refs[2 + 2 * PIPE:2 + 3 * PIPE]
  exr = refs[2 + 3 * PIPE:2 + 4 * PIPE]
  mv = refs[2 + 4 * PIPE]
  den_sp = refs[3 + 4 * PIPE]
  sems = refs[4 + 4 * PIPE:]
  gsem = [sems[5 * j:5 * j + 3] for j in range(PIPE)]
  osem = [sems[5 * j + 3:5 * j + 5] for j in range(PIPE)]

  cid = lax.axis_index("c")
  sid = lax.axis_index("s")
  wid = cid * NS + sid
  pltpu.sync_copy(z16_h.at[pl.ds(sid * ROWS_PT, ROWS_PT)],
                  den_sp.at[pl.ds(sid * ROWS_PT, ROWS_PT)])
  pltpu.sync_copy(mv_h, mv)
  pltpu.sync_copy(src_h.at[wid], srcall)
  pltpu.sync_copy(dst_h.at[wid], dstall)
  plsc.subcore_barrier()
  mrow = mv[...]
  base0 = wid * EPW

  @pl.loop(0, NBODY)
  def _body(k):
    blk0 = k * PIPE
    gds = []
    for j in range(PIPE):
      blk = blk0 + j
      gds.append((
          pltpu.async_copy(asrc_h.at[srcall.at[blk]], asr[j], gsem[j][0]),
          pltpu.async_copy(adst_h.at[dstall.at[blk]], adr[j], gsem[j][1]),
          pltpu.async_copy(ae_h.at[pl.ds(base0 + blk * BLK, BLK)], aer[j],
                           gsem[j][2]),
      ))
    ods = []
    for j in range(PIPE):
      blk = blk0 + j
      for g in gds[j]:
        g.wait()

      @pl.loop(0, BLK)
      def _edge(e):
        a = asr[j][e, :] + adr[j][e, :] + aer[j][e, :]
        a = jnp.where(a < 0.0, a * 0.2, a)
        exr[j][e, :] = jnp.exp(a - mrow)

      ods.append(pltpu.async_copy(
          exr[j], ex_h.at[pl.ds(base0 + blk * BLK, BLK)], osem[j][0]))
      ods.append(pltpu.async_copy(
          exr[j], den_sp.at[dstall.at[blk]], osem[j][1], add=True))
    for o in ods:
      o.wait()

  plsc.subcore_barrier()
  pltpu.sync_copy(den_sp.at[pl.ds(sid * ROWS_PT, ROWS_PT)],
                  den_h.at[cid].at[pl.ds(sid * ROWS_PT, ROWS_PT)])


def _make_sc_pass_c(heads):
  @functools.partial(
      pl.kernel,
      mesh=_MESH,
      out_type=jax.ShapeDtypeStruct((NC, NP, HID), jnp.float32),
      compiler_params=pltpu.CompilerParams(use_tc_tiling_on_sc=False),
      scratch_types=(
          [pltpu.VMEM((NBLK, BLK), jnp.int32) for _ in range(2)]
          + [pltpu.VMEM((BLK, HP), jnp.float32) for _ in range(2 * PIPE_C)]
          + [pltpu.VMEM((BLK, HID), jnp.float32) for _ in range(PIPE_C)]
          + [pltpu.VMEM_SHARED((NP, HID), jnp.float32)]
          + [pltpu.SemaphoreType.DMA for _ in range(4 * PIPE_C)]
      ),
  )
  def _sc_pass_c(src_h, dst_h, ex_h, ivd_h, hw_h, z128_h,
                 agg_h, *refs):
    srcall, dstall = refs[0], refs[1]
    exr = refs[2:2 + PIPE_C]
    ivr = refs[2 + PIPE_C:2 + 2 * PIPE_C]
    rows = refs[2 + 2 * PIPE_C:2 + 3 * PIPE_C]
    agg_sp = refs[2 + 3 * PIPE_C]
    sems = refs[3 + 3 * PIPE_C:]
    gsem = [sems[4 * j:4 * j + 3] for j in range(PIPE_C)]
    osem = [sems[4 * j + 3] for j in range(PIPE_C)]

    cid = lax.axis_index("c")
    sid = lax.axis_index("s")
    wid = cid * NS + sid
    pltpu.sync_copy(z128_h.at[pl.ds(sid * ROWS_PT, ROWS_PT)],
                    agg_sp.at[pl.ds(sid * ROWS_PT, ROWS_PT)])
    pltpu.sync_copy(src_h.at[wid], srcall)
    pltpu.sync_copy(dst_h.at[wid], dstall)
    plsc.subcore_barrier()
    base0 = wid * EPW

    @pl.loop(0, NBODY_C)
    def _body(k):
      blk0 = k * PIPE_C
      gds = []
      for j in range(PIPE_C):
        blk = blk0 + j
        gds.append((
            pltpu.async_copy(ex_h.at[pl.ds(base0 + blk * BLK, BLK)], exr[j],
                             gsem[j][0]),
            pltpu.async_copy(ivd_h.at[dstall.at[blk]], ivr[j], gsem[j][1]),
            pltpu.async_copy(hw_h.at[srcall.at[blk]], rows[j], gsem[j][2]),
        ))
      ods = []
      for j in range(PIPE_C):
        blk = blk0 + j
        for g in gds[j]:
          g.wait()

        @pl.loop(0, BLK)
        def _edge(e):
          crow = exr[j][e, :] * ivr[j][e, :]
          if heads > 1:
            for h in range(heads):
              sc = jnp.take(crow, jnp.full((16,), h, jnp.int32))
              rows[j][e, pl.ds(h * 16, 16)] = rows[j][e, pl.ds(h * 16, 16)] * sc
          else:
            sc = jnp.take(crow, jnp.full((16,), 0, jnp.int32))
            for q in range(HID // 16):
              rows[j][e, pl.ds(q * 16, 16)] = rows[j][e, pl.ds(q * 16, 16)] * sc

        ods.append(pltpu.async_copy(
            rows[j], agg_sp.at[dstall.at[blk]], osem[j], add=True))
      for o in ods:
        o.wait()

    plsc.subcore_barrier()
    pltpu.sync_copy(agg_sp.at[pl.ds(sid * ROWS_PT, ROWS_PT)],
                    agg_h.at[cid].at[pl.ds(sid * ROWS_PT, ROWS_PT)])

  return _sc_pass_c


_SC_PASS_C = {h: _make_sc_pass_c(h) for h in (8, 1)}


def _fold_params(params):
  folded = {}
  p_cols = []
  c_cols = []
  for i, (out, heads, _) in enumerate(LAYER_CFG):
    we = params[f'l{i}_We'].reshape(HID, heads, out)
    m = jnp.einsum('khc,hc->kh', we, params[f'l{i}_att_edge'])
    p_pad = jnp.zeros((16, HP), jnp.float32)
    p_pad = p_pad.at[:, :heads].set(params['edge_proj_W'] @ m)
    c_pad = jnp.zeros((HP,), jnp.float32)
    c_pad = c_pad.at[:heads].set(params['edge_proj_b'] @ m)
    p_cols.append(p_pad)
    c_cols.append(c_pad)

    eye = jnp.eye(heads, dtype=jnp.float32)
    for name in ('src', 'dst'):
      att = params[f'l{i}_att_{name}']
      s = (att[:, :, None] * eye[:, None, :]).reshape(heads * out, heads)
      s_pad = jnp.zeros((HID, HP), jnp.float32)
      s_pad = s_pad.at[:, :heads].set(s)
      folded[f'l{i}_S{name}'] = s_pad
  folded['P_all'] = jnp.concatenate(p_cols, axis=1)
  folded['c_all'] = jnp.concatenate(c_cols, axis=0)
  return folded


def kernel(x, edge_index, edge_attr, params):
  src = edge_index[0].astype(jnp.int32).reshape(NW, NBLK, BLK)
  dst = edge_index[1].astype(jnp.int32).reshape(NW, NBLK, BLK)
  folded = _fold_params(params)

  ae0, ae1, ae2, am0, am1, am2 = _edge_logits(
      edge_attr, folded['P_all'], folded['c_all'])
  ae_list = (ae0, ae1, ae2)
  ae_max = (am0, am1, am2)

  zeros16 = jnp.zeros((NP, HP), jnp.float32)
  zeros128 = jnp.zeros((NP, HID), jnp.float32)

  h, hw, asrc, adst, ms, md = _proj_dense(
      x, params['node_proj_W'], params['node_proj_b'],
      params['l0_W'], folded['l0_Ssrc'], folded['l0_Sdst'])

  for i, (out, heads, concat) in enumerate(LAYER_CFG):
    mvec = jnp.maximum(ms[0] + md[0] + ae_max[i][0], 0.0)
    ex, den_parts = _sc_pass_a(src, dst, asrc, adst, ae_list[i],
                               mvec, zeros16)
    invd = _inv_denom(den_parts)
    agg_parts = _SC_PASS_C[heads](src, dst, ex, invd, hw, zeros128)
    if i < len(LAYER_CFG) - 1:
      h, hw, asrc, adst, ms, md = _epi_dense(
          agg_parts, params[f'l{i}_bias'], params[f'l{i}_bn_gamma'],
          params[f'l{i}_bn_beta'], h,
          params[f'l{i+1}_W'], folded[f'l{i+1}_Ssrc'],
          folded[f'l{i+1}_Sdst'])
    else:
      h = _epilogue(agg_parts, params[f'l{i}_bias'],
                    params[f'l{i}_bn_gamma'], params[f'l{i}_bn_beta'], h,
                    relu=False)
  return h
